# baseline (speedup 1.0000x reference)
import functools

import jax
import jax.numpy as jnp
from jax import lax
from jax.experimental import pallas as pl
from jax.experimental.pallas import tpu as pltpu
from jax.experimental.pallas import tpu_sc as plsc

BATCH = 4096
HIST = 200
HPAD = 208
HC = HPAD // 2
RG = HPAD // 16
DIM = 64
OUT = 256
VOCAB = 1000000
NCORES = 2
NSUB = 16
NW = NCORES * NSUB
BPW = BATCH // NW
LANES = 16
CG = DIM // LANES
VPAIR = VOCAB // 2

_mesh = plsc.VectorSubcoreMesh(core_axis_name="c", subcore_axis_name="s")


def _pairize(table_t4):
    def body(t_ref, y_ref):
        x = t_ref[...].reshape(DIM, 8, 1000)
        for s in range(8):
            t = jnp.swapaxes(x[:, s, :], 0, 1)
            y_ref[pl.ds(s * 500, 500), :] = jnp.concatenate(
                [t[0:500], t[500:1000]], axis=1)

    return pl.pallas_call(
        body,
        grid=(125,),
        in_specs=[pl.BlockSpec((DIM, 1, 8, 1000), lambda i: (0, i, 0, 0))],
        out_specs=pl.BlockSpec((4000, 2 * DIM), lambda i: (i, 0)),
        out_shape=jax.ShapeDtypeStruct((VPAIR, 2 * DIM), jnp.float32),
    )(table_t4)


@functools.partial(
    pl.kernel,
    out_type=jax.ShapeDtypeStruct((BATCH // 2, 2 * DIM), jnp.float32),
    mesh=_mesh,
    compiler_params=pltpu.CompilerParams(use_tc_tiling_on_sc=True),
    scratch_types=[
        pltpu.VMEM((2 * BPW, HC), jnp.int32),
        pltpu.VMEM((BPW, HPAD), jnp.int32),
        pltpu.VMEM((2, HPAD, 2 * DIM), jnp.float32),
        pltpu.VMEM((BPW // 2, 2 * DIM), jnp.float32),
        pltpu.SemaphoreType.DMA,
        pltpu.SemaphoreType.DMA,
    ],
)
def _gather_max(pair_hbm, par_hbm, table_hbm, out_hbm,
                idx_v, par_v, rows_v, pooled_v, sem0, sem1):
    wid = lax.axis_index("s") * NCORES + lax.axis_index("c")
    sems = (sem0, sem1)

    pltpu.sync_copy(pair_hbm.at[pl.ds(wid * 2 * BPW, 2 * BPW)], idx_v)
    pltpu.sync_copy(par_hbm.at[pl.ds(wid * BPW, BPW)], par_v)

    def fire(b, buf):
        for j in range(2):
            pltpu.async_copy(
                table_hbm.at[idx_v.at[2 * b + j]],
                rows_v.at[buf, pl.ds(j * HC, HC)], sems[buf])

    def wait_buf(b, buf):
        for j in range(2):
            pltpu.make_async_copy(
                table_hbm.at[idx_v.at[2 * b + j]],
                rows_v.at[buf, pl.ds(j * HC, HC)], sems[buf]).wait()

    fire(0, 0)
    fire(1, 1)

    @pl.loop(0, BPW, step=2)
    def _pipeline(g):
        for d in range(2):
            b = g + d
            wait_buf(b, d)

            neg = jnp.full((LANES,), -jnp.inf, dtype=jnp.float32)

            def reduce_group(rg, a):
                pvec = par_v[b, pl.ds(rg * LANES, LANES)]
                for l in range(LANES):
                    r = rg * LANES + l
                    p = pvec[l]
                    sel = []
                    for c in range(CG):
                        lo = rows_v[d, r, pl.ds(c * LANES, LANES)]
                        hi = rows_v[d, r, pl.ds(DIM + c * LANES, LANES)]
                        sel.append(jnp.where(p == 1, hi, lo))
                    a = tuple(jnp.maximum(a[c], sel[c]) for c in range(CG))
                return a

            acc = lax.fori_loop(0, RG, reduce_group, (neg, neg, neg, neg))
            for c in range(CG):
                pooled_v[g // 2, pl.ds(d * DIM + c * LANES, LANES)] = acc[c]

            nb = b + 2

            @pl.when(nb < BPW)
            def _():
                fire(nb, d)

    pltpu.sync_copy(pooled_v, out_hbm.at[pl.ds(wid * (BPW // 2), BPW // 2)])


def _linear(pooled, W, b2):
    blk = 512
    grid = BATCH // blk

    def body(p_ref, w_ref, b_ref, o_ref):
        o_ref[...] = lax.dot_general(
            p_ref[...], w_ref[...], (((1,), (1,)), ((), ())),
            preferred_element_type=jnp.float32) + b_ref[...]

    return pl.pallas_call(
        body,
        grid=(grid,),
        in_specs=[
            pl.BlockSpec((blk, DIM), lambda i: (i, 0)),
            pl.BlockSpec((OUT, DIM), lambda i: (0, 0)),
            pl.BlockSpec((1, OUT), lambda i: (0, 0)),
        ],
        out_specs=pl.BlockSpec((blk, OUT), lambda i: (i, 0)),
        out_shape=jax.ShapeDtypeStruct((BATCH, OUT), jnp.float32),
    )(pooled, W, b2)


@jax.jit
def kernel(text, table, W, b):
    text = text.astype(jnp.int32)
    pad = jnp.broadcast_to(text[:, :1], (BATCH, HPAD - HIST))
    text_p = jnp.concatenate([text, pad], axis=1)
    pair = ((text_p // 1000) * 500 + text_p % 500).reshape(2 * BATCH, HC)
    par = (text_p // 500) & 1
    table_pairs = _pairize(jnp.swapaxes(table, 0, 1).reshape(DIM, 125, 8, 1000))
    pooled2 = _gather_max(pair, par, table_pairs)
    pooled = pooled2.reshape(BATCH, DIM)
    return _linear(pooled, W, b.reshape(1, OUT))


# speedup vs baseline: 1.0607x; 1.0607x over previous
import functools

import jax
import jax.numpy as jnp
from jax import lax
from jax.experimental import pallas as pl
from jax.experimental.pallas import tpu as pltpu
from jax.experimental.pallas import tpu_sc as plsc

BATCH = 4096
HIST = 200
HPAD = 208
HC = HPAD // 2
RG = HPAD // 16
DIM = 64
OUT = 256
VOCAB = 1000000
NCORES = 2
NSUB = 16
NW = NCORES * NSUB
BPW = BATCH // NW
LANES = 16
CG = DIM // LANES
VB = 1024
NVB = -(-VOCAB // VB)
VPAIR = NVB * (VB // 2)

_mesh = plsc.VectorSubcoreMesh(core_axis_name="c", subcore_axis_name="s")


def _pairize(table_t):
    def body(t_ref, y_ref):
        eye = (lax.broadcasted_iota(jnp.int32, (DIM, DIM), 0) ==
               lax.broadcasted_iota(jnp.int32, (DIM, DIM), 1)).astype(jnp.float32)
        t = lax.dot_general(
            t_ref[...], eye, (((0,), (0,)), ((), ())),
            preferred_element_type=jnp.float32)
        y_ref[...] = jnp.concatenate([t[0:VB // 2], t[VB // 2:VB]], axis=1)

    return pl.pallas_call(
        body,
        grid=(NVB,),
        in_specs=[pl.BlockSpec((DIM, VB), lambda i: (0, i))],
        out_specs=pl.BlockSpec((VB // 2, 2 * DIM), lambda i: (i, 0)),
        out_shape=jax.ShapeDtypeStruct((VPAIR, 2 * DIM), jnp.float32),
    )(table_t)


@functools.partial(
    pl.kernel,
    out_type=jax.ShapeDtypeStruct((BATCH // 2, 2 * DIM), jnp.float32),
    mesh=_mesh,
    compiler_params=pltpu.CompilerParams(use_tc_tiling_on_sc=True),
    scratch_types=[
        pltpu.VMEM((2 * BPW, HC), jnp.int32),
        pltpu.VMEM((BPW, HPAD), jnp.int32),
        pltpu.VMEM((2, HPAD, 2 * DIM), jnp.float32),
        pltpu.VMEM((BPW // 2, 2 * DIM), jnp.float32),
        pltpu.SemaphoreType.DMA,
        pltpu.SemaphoreType.DMA,
    ],
)
def _gather_max(pair_hbm, par_hbm, table_hbm, out_hbm,
                idx_v, par_v, rows_v, pooled_v, sem0, sem1):
    wid = lax.axis_index("s") * NCORES + lax.axis_index("c")
    sems = (sem0, sem1)

    pltpu.sync_copy(pair_hbm.at[pl.ds(wid * 2 * BPW, 2 * BPW)], idx_v)
    pltpu.sync_copy(par_hbm.at[pl.ds(wid * BPW, BPW)], par_v)

    def fire(b, buf):
        for j in range(2):
            pltpu.async_copy(
                table_hbm.at[idx_v.at[2 * b + j]],
                rows_v.at[buf, pl.ds(j * HC, HC)], sems[buf])

    def wait_buf(b, buf):
        for j in range(2):
            pltpu.make_async_copy(
                table_hbm.at[idx_v.at[2 * b + j]],
                rows_v.at[buf, pl.ds(j * HC, HC)], sems[buf]).wait()

    fire(0, 0)
    fire(1, 1)

    @pl.loop(0, BPW, step=2)
    def _pipeline(g):
        for d in range(2):
            b = g + d
            wait_buf(b, d)

            neg = jnp.full((LANES,), -jnp.inf, dtype=jnp.float32)

            def reduce_group(rg, a):
                pvec = par_v[b, pl.ds(rg * LANES, LANES)]
                for l in range(LANES):
                    r = rg * LANES + l
                    p = pvec[l]
                    sel = []
                    for c in range(CG):
                        lo = rows_v[d, r, pl.ds(c * LANES, LANES)]
                        hi = rows_v[d, r, pl.ds(DIM + c * LANES, LANES)]
                        sel.append(jnp.where(p == 1, hi, lo))
                    a = tuple(jnp.maximum(a[c], sel[c]) for c in range(CG))
                return a

            acc = lax.fori_loop(0, RG, reduce_group, (neg, neg, neg, neg))
            for c in range(CG):
                pooled_v[g // 2, pl.ds(d * DIM + c * LANES, LANES)] = acc[c]

            nb = b + 2

            @pl.when(nb < BPW)
            def _():
                fire(nb, d)

    pltpu.sync_copy(pooled_v, out_hbm.at[pl.ds(wid * (BPW // 2), BPW // 2)])


def _linear(pooled, W, b2):
    blk = 512
    grid = BATCH // blk

    def body(p_ref, w_ref, b_ref, o_ref):
        o_ref[...] = lax.dot_general(
            p_ref[...], w_ref[...], (((1,), (1,)), ((), ())),
            preferred_element_type=jnp.float32) + b_ref[...]

    return pl.pallas_call(
        body,
        grid=(grid,),
        in_specs=[
            pl.BlockSpec((blk, DIM), lambda i: (i, 0)),
            pl.BlockSpec((OUT, DIM), lambda i: (0, 0)),
            pl.BlockSpec((1, OUT), lambda i: (0, 0)),
        ],
        out_specs=pl.BlockSpec((blk, OUT), lambda i: (i, 0)),
        out_shape=jax.ShapeDtypeStruct((BATCH, OUT), jnp.float32),
    )(pooled, W, b2)


@jax.jit
def kernel(text, table, W, b):
    text = text.astype(jnp.int32)
    pad = jnp.broadcast_to(text[:, :1], (BATCH, HPAD - HIST))
    text_p = jnp.concatenate([text, pad], axis=1)
    pair = (((text_p >> 10) << 9) + (text_p & 511)).reshape(2 * BATCH, HC)
    par = (text_p >> 9) & 1
    table_pairs = _pairize(jnp.swapaxes(table, 0, 1))
    pooled2 = _gather_max(pair, par, table_pairs)
    pooled = pooled2.reshape(BATCH, DIM)
    return _linear(pooled, W, b.reshape(1, OUT))


# speedup vs baseline: 1.0968x; 1.0341x over previous
import functools

import jax
import jax.numpy as jnp
from jax import lax
from jax.experimental import pallas as pl
from jax.experimental.pallas import tpu as pltpu
from jax.experimental.pallas import tpu_sc as plsc

BATCH = 4096
HIST = 200
HPAD = 208
HC = HPAD // 2
RG = HPAD // 16
DIM = 64
OUT = 256
VOCAB = 1000000
NCORES = 2
NSUB = 16
NW = NCORES * NSUB
BPW = BATCH // NW
LANES = 16
CG = DIM // LANES
VB = 1024
NVB = -(-VOCAB // VB)
VPAIR = NVB * (VB // 2)

_mesh = plsc.VectorSubcoreMesh(core_axis_name="c", subcore_axis_name="s")


def _pairize(table_t):
    def body(t_ref, y_ref):
        t = jnp.swapaxes(t_ref[...], 0, 1)
        y_ref[...] = jnp.concatenate([t[0:VB // 2], t[VB // 2:VB]], axis=1)

    return pl.pallas_call(
        body,
        grid=(NVB,),
        in_specs=[pl.BlockSpec((DIM, VB), lambda i: (0, i))],
        out_specs=pl.BlockSpec((VB // 2, 2 * DIM), lambda i: (i, 0)),
        out_shape=jax.ShapeDtypeStruct((VPAIR, 2 * DIM), jnp.float32),
    )(table_t)


@functools.partial(
    pl.kernel,
    out_type=jax.ShapeDtypeStruct((BATCH // 2, 2 * DIM), jnp.float32),
    mesh=_mesh,
    compiler_params=pltpu.CompilerParams(use_tc_tiling_on_sc=True),
    scratch_types=[
        pltpu.VMEM((2 * BPW, HC), jnp.int32),
        pltpu.VMEM((BPW, HPAD), jnp.int32),
        pltpu.VMEM((2, HPAD, 2 * DIM), jnp.float32),
        pltpu.VMEM((BPW // 2, 2 * DIM), jnp.float32),
        pltpu.SemaphoreType.DMA,
        pltpu.SemaphoreType.DMA,
    ],
)
def _gather_max(pair_hbm, par_hbm, table_hbm, out_hbm,
                idx_v, par_v, rows_v, pooled_v, sem0, sem1):
    wid = lax.axis_index("s") * NCORES + lax.axis_index("c")
    sems = (sem0, sem1)

    pltpu.sync_copy(pair_hbm.at[pl.ds(wid * 2 * BPW, 2 * BPW)], idx_v)
    pltpu.sync_copy(par_hbm.at[pl.ds(wid * BPW, BPW)], par_v)

    def fire(b, buf):
        for j in range(2):
            pltpu.async_copy(
                table_hbm.at[idx_v.at[2 * b + j]],
                rows_v.at[buf, pl.ds(j * HC, HC)], sems[buf])

    def wait_buf(b, buf):
        for j in range(2):
            pltpu.make_async_copy(
                table_hbm.at[idx_v.at[2 * b + j]],
                rows_v.at[buf, pl.ds(j * HC, HC)], sems[buf]).wait()

    fire(0, 0)
    fire(1, 1)

    @pl.loop(0, BPW, step=2)
    def _pipeline(g):
        for d in range(2):
            b = g + d
            wait_buf(b, d)

            neg = jnp.full((LANES,), -jnp.inf, dtype=jnp.float32)

            def reduce_group(rg, a):
                pvec = par_v[b, pl.ds(rg * LANES, LANES)]
                for l in range(LANES):
                    r = rg * LANES + l
                    p = pvec[l]
                    sel = []
                    for c in range(CG):
                        lo = rows_v[d, r, pl.ds(c * LANES, LANES)]
                        hi = rows_v[d, r, pl.ds(DIM + c * LANES, LANES)]
                        sel.append(jnp.where(p == 1, hi, lo))
                    a = tuple(jnp.maximum(a[c], sel[c]) for c in range(CG))
                return a

            acc = lax.fori_loop(0, RG, reduce_group, (neg, neg, neg, neg))
            for c in range(CG):
                pooled_v[g // 2, pl.ds(d * DIM + c * LANES, LANES)] = acc[c]

            nb = b + 2

            @pl.when(nb < BPW)
            def _():
                fire(nb, d)

    pltpu.sync_copy(pooled_v, out_hbm.at[pl.ds(wid * (BPW // 2), BPW // 2)])


def _linear(pooled, W, b2):
    blk = 512
    grid = BATCH // blk

    def body(p_ref, w_ref, b_ref, o_ref):
        o_ref[...] = lax.dot_general(
            p_ref[...], w_ref[...], (((1,), (1,)), ((), ())),
            preferred_element_type=jnp.float32) + b_ref[...]

    return pl.pallas_call(
        body,
        grid=(grid,),
        in_specs=[
            pl.BlockSpec((blk, DIM), lambda i: (i, 0)),
            pl.BlockSpec((OUT, DIM), lambda i: (0, 0)),
            pl.BlockSpec((1, OUT), lambda i: (0, 0)),
        ],
        out_specs=pl.BlockSpec((blk, OUT), lambda i: (i, 0)),
        out_shape=jax.ShapeDtypeStruct((BATCH, OUT), jnp.float32),
    )(pooled, W, b2)


@jax.jit
def kernel(text, table, W, b):
    text = text.astype(jnp.int32)
    pad = jnp.broadcast_to(text[:, :1], (BATCH, HPAD - HIST))
    text_p = jnp.concatenate([text, pad], axis=1)
    pair = (((text_p >> 10) << 9) + (text_p & 511)).reshape(2 * BATCH, HC)
    par = (text_p >> 9) & 1
    table_pairs = _pairize(jnp.swapaxes(table, 0, 1))
    pooled2 = _gather_max(pair, par, table_pairs)
    pooled = pooled2.reshape(BATCH, DIM)
    return _linear(pooled, W, b.reshape(1, OUT))


# speedup vs baseline: 1.7791x; 1.6221x over previous
import functools

import jax
import jax.numpy as jnp
from jax import lax
from jax.experimental import pallas as pl
from jax.experimental.pallas import tpu as pltpu
from jax.experimental.pallas import tpu_sc as plsc

BATCH = 4096
HIST = 200
HPAD = 208
HC = HPAD // 2
RG = HPAD // 16
DIM = 64
OUT = 256
VOCAB = 1000000
NCORES = 2
NSUB = 16
NW = NCORES * NSUB
BPW = BATCH // NW
LANES = 16
CG = DIM // LANES
VB = 4096
NVB = -(-VOCAB // VB)
VPAIR = NVB * (VB // 2)

_mesh = plsc.VectorSubcoreMesh(core_axis_name="c", subcore_axis_name="s")


def _pairize(table_t):
    def body(t_ref, y_ref):
        t = jnp.swapaxes(t_ref[...], 0, 1)
        y_ref[...] = jnp.concatenate([t[0:VB // 2], t[VB // 2:VB]], axis=1)

    return pl.pallas_call(
        body,
        grid=(NVB,),
        in_specs=[pl.BlockSpec((DIM, VB), lambda i: (0, i))],
        out_specs=pl.BlockSpec((VB // 2, 2 * DIM), lambda i: (i, 0)),
        out_shape=jax.ShapeDtypeStruct((VPAIR, 2 * DIM), jnp.float32),
    )(table_t)


@functools.partial(
    pl.kernel,
    out_type=jax.ShapeDtypeStruct((BATCH // 2, 2 * DIM), jnp.float32),
    mesh=_mesh,
    compiler_params=pltpu.CompilerParams(use_tc_tiling_on_sc=True),
    scratch_types=[
        pltpu.VMEM((2 * BPW, HC), jnp.int32),
        pltpu.VMEM((BPW, HPAD), jnp.int32),
        pltpu.VMEM((2, HPAD, 2 * DIM), jnp.float32),
        pltpu.VMEM((BPW // 2, 2 * DIM), jnp.float32),
        pltpu.SemaphoreType.DMA,
        pltpu.SemaphoreType.DMA,
    ],
)
def _gather_max(pair_hbm, par_hbm, table_hbm, out_hbm,
                idx_v, par_v, rows_v, pooled_v, sem0, sem1):
    wid = lax.axis_index("s") * NCORES + lax.axis_index("c")
    sems = (sem0, sem1)

    pltpu.sync_copy(pair_hbm.at[pl.ds(wid * 2 * BPW, 2 * BPW)], idx_v)
    pltpu.sync_copy(par_hbm.at[pl.ds(wid * BPW, BPW)], par_v)

    def fire(b, buf):
        for j in range(2):
            pltpu.async_copy(
                table_hbm.at[idx_v.at[2 * b + j]],
                rows_v.at[buf, pl.ds(j * HC, HC)], sems[buf])

    def wait_buf(b, buf):
        for j in range(2):
            pltpu.make_async_copy(
                table_hbm.at[idx_v.at[2 * b + j]],
                rows_v.at[buf, pl.ds(j * HC, HC)], sems[buf]).wait()

    fire(0, 0)
    fire(1, 1)

    @pl.loop(0, BPW, step=2)
    def _pipeline(g):
        for d in range(2):
            b = g + d
            wait_buf(b, d)

            neg = jnp.full((LANES,), -jnp.inf, dtype=jnp.float32)

            def reduce_group(rg, a):
                pvec = par_v[b, pl.ds(rg * LANES, LANES)]
                for l in range(LANES):
                    r = rg * LANES + l
                    p = pvec[l]
                    sel = []
                    for c in range(CG):
                        lo = rows_v[d, r, pl.ds(c * LANES, LANES)]
                        hi = rows_v[d, r, pl.ds(DIM + c * LANES, LANES)]
                        sel.append(jnp.where(p == 1, hi, lo))
                    a = tuple(jnp.maximum(a[c], sel[c]) for c in range(CG))
                return a

            acc = lax.fori_loop(0, RG, reduce_group, (neg, neg, neg, neg))
            for c in range(CG):
                pooled_v[g // 2, pl.ds(d * DIM + c * LANES, LANES)] = acc[c]

            nb = b + 2

            @pl.when(nb < BPW)
            def _():
                fire(nb, d)

    pltpu.sync_copy(pooled_v, out_hbm.at[pl.ds(wid * (BPW // 2), BPW // 2)])


def _linear(pooled, W, b2):
    blk = 512
    grid = BATCH // blk

    def body(p_ref, w_ref, b_ref, o_ref):
        o_ref[...] = lax.dot_general(
            p_ref[...], w_ref[...], (((1,), (1,)), ((), ())),
            preferred_element_type=jnp.float32) + b_ref[...]

    return pl.pallas_call(
        body,
        grid=(grid,),
        in_specs=[
            pl.BlockSpec((blk, DIM), lambda i: (i, 0)),
            pl.BlockSpec((OUT, DIM), lambda i: (0, 0)),
            pl.BlockSpec((1, OUT), lambda i: (0, 0)),
        ],
        out_specs=pl.BlockSpec((blk, OUT), lambda i: (i, 0)),
        out_shape=jax.ShapeDtypeStruct((BATCH, OUT), jnp.float32),
    )(pooled, W, b2)


@jax.jit
def kernel(text, table, W, b):
    text = text.astype(jnp.int32)
    pad = jnp.broadcast_to(text[:, :1], (BATCH, HPAD - HIST))
    text_p = jnp.concatenate([text, pad], axis=1)
    pair = (((text_p >> 12) << 11) + (text_p & 2047)).reshape(2 * BATCH, HC)
    par = (text_p >> 11) & 1
    table_pairs = _pairize(jnp.swapaxes(table, 0, 1))
    pooled2 = _gather_max(pair, par, table_pairs)
    pooled = pooled2.reshape(BATCH, DIM)
    return _linear(pooled, W, b.reshape(1, OUT))


# speedup vs baseline: 2.0839x; 1.1713x over previous
import functools

import jax
import jax.numpy as jnp
from jax import lax
from jax.experimental import pallas as pl
from jax.experimental.pallas import tpu as pltpu
from jax.experimental.pallas import tpu_sc as plsc

BATCH = 4096
HIST = 200
HPAD = 208
HC = HPAD // 2
DIM = 64
OUT = 256
VOCAB = 1000000
NCORES = 2
NSUB = 16
NW = NCORES * NSUB
BPW = BATCH // NW
LANES = 16
CG = DIM // LANES
VB = 4096
NVB = -(-VOCAB // VB)
VPAIR = NVB * (VB // 2)

_mesh = plsc.VectorSubcoreMesh(core_axis_name="c", subcore_axis_name="s")


def _pairize(table_t):
    def body(t_ref, y_ref):
        t = jnp.swapaxes(t_ref[...], 0, 1)
        y_ref[...] = jnp.concatenate([t[0:VB // 2], t[VB // 2:VB]], axis=1)

    return pl.pallas_call(
        body,
        grid=(NVB,),
        in_specs=[pl.BlockSpec((DIM, VB), lambda i: (0, i))],
        out_specs=pl.BlockSpec((VB // 2, 2 * DIM), lambda i: (i, 0)),
        out_shape=jax.ShapeDtypeStruct((VPAIR, 2 * DIM), jnp.float32),
    )(table_t)


@functools.partial(
    pl.kernel,
    out_type=jax.ShapeDtypeStruct((BATCH, DIM), jnp.float32),
    mesh=_mesh,
    compiler_params=pltpu.CompilerParams(use_tc_tiling_on_sc=False),
    scratch_types=[
        pltpu.VMEM((2 * BPW, HC), jnp.int32),
        pltpu.VMEM((2, HPAD, DIM), jnp.float32),
        pltpu.VMEM((BPW, DIM), jnp.float32),
        pltpu.SemaphoreType.DMA,
        pltpu.SemaphoreType.DMA,
    ],
)
def _gather_max(text_hbm, table_hbm, out_hbm, idx_v, rows_v, pooled_v, sem0, sem1):
    wid = lax.axis_index("s") * NCORES + lax.axis_index("c")
    base = wid * (2 * BPW)
    sems = (sem0, sem1)

    pltpu.sync_copy(text_hbm.at[pl.ds(base, 2 * BPW)], idx_v)

    def fire(b, buf):
        pltpu.async_copy(
            table_hbm.at[idx_v.at[2 * b]], rows_v.at[buf, pl.ds(0, HC)], sems[buf])
        pltpu.async_copy(
            table_hbm.at[idx_v.at[2 * b + 1]], rows_v.at[buf, pl.ds(HC, HC)], sems[buf])

    def wait_buf(b, buf):
        pltpu.make_async_copy(
            table_hbm.at[idx_v.at[2 * b]], rows_v.at[buf, pl.ds(0, HC)], sems[buf]).wait()
        pltpu.make_async_copy(
            table_hbm.at[idx_v.at[2 * b + 1]], rows_v.at[buf, pl.ds(HC, HC)], sems[buf]).wait()

    fire(0, 0)
    fire(1, 1)

    @pl.loop(0, BPW, step=2)
    def _pipeline(g):
        for d in range(2):
            b = g + d
            wait_buf(b, d)

            def reduce_row(r, acc):
                return tuple(
                    jnp.maximum(acc[c], rows_v[d, r, pl.ds(c * LANES, LANES)])
                    for c in range(CG))

            acc0 = tuple(rows_v[d, 0, pl.ds(c * LANES, LANES)] for c in range(CG))
            acc = lax.fori_loop(1, HPAD, reduce_row, acc0)
            for c in range(CG):
                pooled_v[b, pl.ds(c * LANES, LANES)] = acc[c]

            nb = b + 2

            @pl.when(nb < BPW)
            def _():
                fire(nb, d)

    pltpu.sync_copy(pooled_v, out_hbm.at[pl.ds(wid * BPW, BPW)])


def _linear(pooled, W, b2):
    blk = 512
    grid = BATCH // blk

    def body(p_ref, w_ref, b_ref, o_ref):
        o_ref[...] = lax.dot_general(
            p_ref[...], w_ref[...], (((1,), (1,)), ((), ())),
            preferred_element_type=jnp.float32) + b_ref[...]

    return pl.pallas_call(
        body,
        grid=(grid,),
        in_specs=[
            pl.BlockSpec((blk, DIM), lambda i: (i, 0)),
            pl.BlockSpec((OUT, DIM), lambda i: (0, 0)),
            pl.BlockSpec((1, OUT), lambda i: (0, 0)),
        ],
        out_specs=pl.BlockSpec((blk, OUT), lambda i: (i, 0)),
        out_shape=jax.ShapeDtypeStruct((BATCH, OUT), jnp.float32),
    )(pooled, W, b2)


@jax.jit
def kernel(text, table, W, b):
    text = text.astype(jnp.int32)
    pad = jnp.broadcast_to(text[:, :1], (BATCH, HPAD - HIST))
    text_p = jnp.concatenate([text, pad], axis=1)
    flat = ((((text_p >> 12) << 11) + (text_p & 2047)) << 1) + ((text_p >> 11) & 1)
    flat = flat.reshape(2 * BATCH, HC)
    table_pairs = _pairize(jnp.swapaxes(table, 0, 1))
    table_rows = table_pairs.reshape(2 * VPAIR, DIM)
    pooled = _gather_max(flat, table_rows)
    return _linear(pooled, W, b.reshape(1, OUT))


# speedup vs baseline: 2.4109x; 1.1569x over previous
import functools

import jax
import jax.numpy as jnp
from jax import lax
from jax.experimental import pallas as pl
from jax.experimental.pallas import tpu as pltpu
from jax.experimental.pallas import tpu_sc as plsc

BATCH = 4096
HIST = 200
HPAD = 208
HC = HPAD // 2
DIM = 64
OUT = 256
VOCAB = 1000000
NCORES = 2
NSUB = 16
NW = NCORES * NSUB
BPW = BATCH // NW
LANES = 16
CG = DIM // LANES
VB = 8192
NVB = -(-VOCAB // VB)
VPAIR = NVB * (VB // 2)

_mesh = plsc.VectorSubcoreMesh(core_axis_name="c", subcore_axis_name="s")


def _pairize(table_t):
    def body(t_ref, y_ref):
        t = jnp.swapaxes(t_ref[...], 0, 1)
        y_ref[...] = jnp.concatenate([t[0:VB // 2], t[VB // 2:VB]], axis=1)

    return pl.pallas_call(
        body,
        grid=(NVB,),
        in_specs=[pl.BlockSpec((DIM, VB), lambda i: (0, i))],
        out_specs=pl.BlockSpec((VB // 2, 2 * DIM), lambda i: (i, 0)),
        out_shape=jax.ShapeDtypeStruct((VPAIR, 2 * DIM), jnp.float32),
    )(table_t)


@functools.partial(
    pl.kernel,
    out_type=jax.ShapeDtypeStruct((BATCH, DIM), jnp.float32),
    mesh=_mesh,
    compiler_params=pltpu.CompilerParams(use_tc_tiling_on_sc=False),
    scratch_types=[
        pltpu.VMEM((2 * BPW, HC), jnp.int32),
        pltpu.VMEM((2, HPAD, DIM), jnp.float32),
        pltpu.VMEM((BPW, DIM), jnp.float32),
        pltpu.SemaphoreType.DMA,
        pltpu.SemaphoreType.DMA,
    ],
)
def _gather_max(text_hbm, table_hbm, out_hbm, idx_v, rows_v, pooled_v, sem0, sem1):
    wid = lax.axis_index("s") * NCORES + lax.axis_index("c")
    base = wid * (2 * BPW)
    sems = (sem0, sem1)

    pltpu.sync_copy(text_hbm.at[pl.ds(base, 2 * BPW)], idx_v)

    def fire(b, buf):
        pltpu.async_copy(
            table_hbm.at[idx_v.at[2 * b]], rows_v.at[buf, pl.ds(0, HC)], sems[buf])
        pltpu.async_copy(
            table_hbm.at[idx_v.at[2 * b + 1]], rows_v.at[buf, pl.ds(HC, HC)], sems[buf])

    def wait_buf(b, buf):
        pltpu.make_async_copy(
            table_hbm.at[idx_v.at[2 * b]], rows_v.at[buf, pl.ds(0, HC)], sems[buf]).wait()
        pltpu.make_async_copy(
            table_hbm.at[idx_v.at[2 * b + 1]], rows_v.at[buf, pl.ds(HC, HC)], sems[buf]).wait()

    fire(0, 0)
    fire(1, 1)

    @pl.loop(0, BPW, step=2)
    def _pipeline(g):
        for d in range(2):
            b = g + d
            wait_buf(b, d)

            def reduce_row(r, acc):
                return tuple(
                    jnp.maximum(acc[c], rows_v[d, r, pl.ds(c * LANES, LANES)])
                    for c in range(CG))

            acc0 = tuple(rows_v[d, 0, pl.ds(c * LANES, LANES)] for c in range(CG))
            acc = lax.fori_loop(1, HPAD, reduce_row, acc0)
            for c in range(CG):
                pooled_v[b, pl.ds(c * LANES, LANES)] = acc[c]

            nb = b + 2

            @pl.when(nb < BPW)
            def _():
                fire(nb, d)

    pltpu.sync_copy(pooled_v, out_hbm.at[pl.ds(wid * BPW, BPW)])


def _linear(pooled, W, b2):
    blk = 512
    grid = BATCH // blk

    def body(p_ref, w_ref, b_ref, o_ref):
        o_ref[...] = lax.dot_general(
            p_ref[...], w_ref[...], (((1,), (1,)), ((), ())),
            preferred_element_type=jnp.float32) + b_ref[...]

    return pl.pallas_call(
        body,
        grid=(grid,),
        in_specs=[
            pl.BlockSpec((blk, DIM), lambda i: (i, 0)),
            pl.BlockSpec((OUT, DIM), lambda i: (0, 0)),
            pl.BlockSpec((1, OUT), lambda i: (0, 0)),
        ],
        out_specs=pl.BlockSpec((blk, OUT), lambda i: (i, 0)),
        out_shape=jax.ShapeDtypeStruct((BATCH, OUT), jnp.float32),
    )(pooled, W, b2)


@jax.jit
def kernel(text, table, W, b):
    text = text.astype(jnp.int32)
    pad = jnp.broadcast_to(text[:, :1], (BATCH, HPAD - HIST))
    text_p = jnp.concatenate([text, pad], axis=1)
    flat = ((((text_p >> 13) << 12) + (text_p & 4095)) << 1) + ((text_p >> 12) & 1)
    flat = flat.reshape(2 * BATCH, HC)
    table_pairs = _pairize(jnp.swapaxes(table, 0, 1))
    table_rows = table_pairs.reshape(2 * VPAIR, DIM)
    pooled = _gather_max(flat, table_rows)
    return _linear(pooled, W, b.reshape(1, OUT))


# speedup vs baseline: 2.6162x; 1.0852x over previous
import functools

import jax
import jax.numpy as jnp
from jax import lax
from jax.experimental import pallas as pl
from jax.experimental.pallas import tpu as pltpu
from jax.experimental.pallas import tpu_sc as plsc

BATCH = 4096
HIST = 200
HPAD = 208
HC = HPAD // 2
DIM = 64
OUT = 256
VOCAB = 1000000
NCORES = 2
NSUB = 16
NW = NCORES * NSUB
BPW = BATCH // NW
LANES = 16
CG = DIM // LANES
VB = 16384
NVB = -(-VOCAB // VB)
VPAIR = NVB * (VB // 2)

_mesh = plsc.VectorSubcoreMesh(core_axis_name="c", subcore_axis_name="s")


def _pairize(table_t):
    def body(t_ref, y_ref):
        t = jnp.swapaxes(t_ref[...], 0, 1)
        y_ref[...] = jnp.concatenate([t[0:VB // 2], t[VB // 2:VB]], axis=1)

    return pl.pallas_call(
        body,
        grid=(NVB,),
        in_specs=[pl.BlockSpec((DIM, VB), lambda i: (0, i))],
        out_specs=pl.BlockSpec((VB // 2, 2 * DIM), lambda i: (i, 0)),
        out_shape=jax.ShapeDtypeStruct((VPAIR, 2 * DIM), jnp.float32),
    )(table_t)


@functools.partial(
    pl.kernel,
    out_type=jax.ShapeDtypeStruct((BATCH, DIM), jnp.float32),
    mesh=_mesh,
    compiler_params=pltpu.CompilerParams(use_tc_tiling_on_sc=False),
    scratch_types=[
        pltpu.VMEM((2 * BPW, HC), jnp.int32),
        pltpu.VMEM((2, HPAD, DIM), jnp.float32),
        pltpu.VMEM((BPW, DIM), jnp.float32),
        pltpu.SemaphoreType.DMA,
        pltpu.SemaphoreType.DMA,
    ],
)
def _gather_max(text_hbm, table_hbm, out_hbm, idx_v, rows_v, pooled_v, sem0, sem1):
    wid = lax.axis_index("s") * NCORES + lax.axis_index("c")
    base = wid * (2 * BPW)
    sems = (sem0, sem1)

    pltpu.sync_copy(text_hbm.at[pl.ds(base, 2 * BPW)], idx_v)

    def fire(b, buf):
        pltpu.async_copy(
            table_hbm.at[idx_v.at[2 * b]], rows_v.at[buf, pl.ds(0, HC)], sems[buf])
        pltpu.async_copy(
            table_hbm.at[idx_v.at[2 * b + 1]], rows_v.at[buf, pl.ds(HC, HC)], sems[buf])

    def wait_buf(b, buf):
        pltpu.make_async_copy(
            table_hbm.at[idx_v.at[2 * b]], rows_v.at[buf, pl.ds(0, HC)], sems[buf]).wait()
        pltpu.make_async_copy(
            table_hbm.at[idx_v.at[2 * b + 1]], rows_v.at[buf, pl.ds(HC, HC)], sems[buf]).wait()

    fire(0, 0)
    fire(1, 1)

    @pl.loop(0, BPW, step=2)
    def _pipeline(g):
        for d in range(2):
            b = g + d
            wait_buf(b, d)

            def reduce_row(r, acc):
                return tuple(
                    jnp.maximum(acc[c], rows_v[d, r, pl.ds(c * LANES, LANES)])
                    for c in range(CG))

            acc0 = tuple(rows_v[d, 0, pl.ds(c * LANES, LANES)] for c in range(CG))
            acc = lax.fori_loop(1, HPAD, reduce_row, acc0)
            for c in range(CG):
                pooled_v[b, pl.ds(c * LANES, LANES)] = acc[c]

            nb = b + 2

            @pl.when(nb < BPW)
            def _():
                fire(nb, d)

    pltpu.sync_copy(pooled_v, out_hbm.at[pl.ds(wid * BPW, BPW)])


def _linear(pooled, W, b2):
    blk = 512
    grid = BATCH // blk

    def body(p_ref, w_ref, b_ref, o_ref):
        o_ref[...] = lax.dot_general(
            p_ref[...], w_ref[...], (((1,), (1,)), ((), ())),
            preferred_element_type=jnp.float32) + b_ref[...]

    return pl.pallas_call(
        body,
        grid=(grid,),
        in_specs=[
            pl.BlockSpec((blk, DIM), lambda i: (i, 0)),
            pl.BlockSpec((OUT, DIM), lambda i: (0, 0)),
            pl.BlockSpec((1, OUT), lambda i: (0, 0)),
        ],
        out_specs=pl.BlockSpec((blk, OUT), lambda i: (i, 0)),
        out_shape=jax.ShapeDtypeStruct((BATCH, OUT), jnp.float32),
    )(pooled, W, b2)


@jax.jit
def kernel(text, table, W, b):
    text = text.astype(jnp.int32)
    pad = jnp.broadcast_to(text[:, :1], (BATCH, HPAD - HIST))
    text_p = jnp.concatenate([text, pad], axis=1)
    flat = ((((text_p >> 14) << 13) + (text_p & 8191)) << 1) + ((text_p >> 13) & 1)
    flat = flat.reshape(2 * BATCH, HC)
    table_pairs = _pairize(jnp.swapaxes(table, 0, 1))
    table_rows = table_pairs.reshape(2 * VPAIR, DIM)
    pooled = _gather_max(flat, table_rows)
    return _linear(pooled, W, b.reshape(1, OUT))


# speedup vs baseline: 2.7180x; 1.0389x over previous
import functools

import jax
import jax.numpy as jnp
from jax import lax
from jax.experimental import pallas as pl
from jax.experimental.pallas import tpu as pltpu
from jax.experimental.pallas import tpu_sc as plsc

BATCH = 4096
HIST = 200
HPAD = 208
HC = HPAD // 2
DIM = 64
OUT = 256
VOCAB = 1000000
NCORES = 2
NSUB = 16
NW = NCORES * NSUB
BPW = BATCH // NW
LANES = 16
CG = DIM // LANES
VB = 32768
NVB = -(-VOCAB // VB)
VPAIR = NVB * (VB // 2)

_mesh = plsc.VectorSubcoreMesh(core_axis_name="c", subcore_axis_name="s")


def _pairize(table_t):
    def body(t_ref, y_ref):
        t = jnp.swapaxes(t_ref[...], 0, 1)
        y_ref[...] = jnp.concatenate([t[0:VB // 2], t[VB // 2:VB]], axis=1)

    return pl.pallas_call(
        body,
        grid=(NVB,),
        in_specs=[pl.BlockSpec((DIM, VB), lambda i: (0, i))],
        out_specs=pl.BlockSpec((VB // 2, 2 * DIM), lambda i: (i, 0)),
        out_shape=jax.ShapeDtypeStruct((VPAIR, 2 * DIM), jnp.float32),
    )(table_t)


@functools.partial(
    pl.kernel,
    out_type=jax.ShapeDtypeStruct((BATCH, DIM), jnp.float32),
    mesh=_mesh,
    compiler_params=pltpu.CompilerParams(use_tc_tiling_on_sc=False),
    scratch_types=[
        pltpu.VMEM((2 * BPW, HC), jnp.int32),
        pltpu.VMEM((2, HPAD, DIM), jnp.float32),
        pltpu.VMEM((BPW, DIM), jnp.float32),
        pltpu.SemaphoreType.DMA,
        pltpu.SemaphoreType.DMA,
    ],
)
def _gather_max(text_hbm, table_hbm, out_hbm, idx_v, rows_v, pooled_v, sem0, sem1):
    wid = lax.axis_index("s") * NCORES + lax.axis_index("c")
    base = wid * (2 * BPW)
    sems = (sem0, sem1)

    pltpu.sync_copy(text_hbm.at[pl.ds(base, 2 * BPW)], idx_v)

    def fire(b, buf):
        pltpu.async_copy(
            table_hbm.at[idx_v.at[2 * b]], rows_v.at[buf, pl.ds(0, HC)], sems[buf])
        pltpu.async_copy(
            table_hbm.at[idx_v.at[2 * b + 1]], rows_v.at[buf, pl.ds(HC, HC)], sems[buf])

    def wait_buf(b, buf):
        pltpu.make_async_copy(
            table_hbm.at[idx_v.at[2 * b]], rows_v.at[buf, pl.ds(0, HC)], sems[buf]).wait()
        pltpu.make_async_copy(
            table_hbm.at[idx_v.at[2 * b + 1]], rows_v.at[buf, pl.ds(HC, HC)], sems[buf]).wait()

    fire(0, 0)
    fire(1, 1)

    @pl.loop(0, BPW, step=2)
    def _pipeline(g):
        for d in range(2):
            b = g + d
            wait_buf(b, d)

            def reduce_row(r, acc):
                return tuple(
                    jnp.maximum(acc[c], rows_v[d, r, pl.ds(c * LANES, LANES)])
                    for c in range(CG))

            acc0 = tuple(rows_v[d, 0, pl.ds(c * LANES, LANES)] for c in range(CG))
            acc = lax.fori_loop(1, HPAD, reduce_row, acc0)
            for c in range(CG):
                pooled_v[b, pl.ds(c * LANES, LANES)] = acc[c]

            nb = b + 2

            @pl.when(nb < BPW)
            def _():
                fire(nb, d)

    pltpu.sync_copy(pooled_v, out_hbm.at[pl.ds(wid * BPW, BPW)])


def _linear(pooled, W, b2):
    blk = 512
    grid = BATCH // blk

    def body(p_ref, w_ref, b_ref, o_ref):
        o_ref[...] = lax.dot_general(
            p_ref[...], w_ref[...], (((1,), (1,)), ((), ())),
            preferred_element_type=jnp.float32) + b_ref[...]

    return pl.pallas_call(
        body,
        grid=(grid,),
        in_specs=[
            pl.BlockSpec((blk, DIM), lambda i: (i, 0)),
            pl.BlockSpec((OUT, DIM), lambda i: (0, 0)),
            pl.BlockSpec((1, OUT), lambda i: (0, 0)),
        ],
        out_specs=pl.BlockSpec((blk, OUT), lambda i: (i, 0)),
        out_shape=jax.ShapeDtypeStruct((BATCH, OUT), jnp.float32),
    )(pooled, W, b2)


@jax.jit
def kernel(text, table, W, b):
    text = text.astype(jnp.int32)
    pad = jnp.broadcast_to(text[:, :1], (BATCH, HPAD - HIST))
    text_p = jnp.concatenate([text, pad], axis=1)
    flat = ((((text_p >> 15) << 14) + (text_p & 16383)) << 1) + ((text_p >> 14) & 1)
    flat = flat.reshape(2 * BATCH, HC)
    table_pairs = _pairize(jnp.swapaxes(table, 0, 1))
    table_rows = table_pairs.reshape(2 * VPAIR, DIM)
    pooled = _gather_max(flat, table_rows)
    return _linear(pooled, W, b.reshape(1, OUT))
